# baseline (speedup 1.0000x reference)
import jax
import jax.numpy as jnp
from jax.experimental import pallas as pl
from jax.experimental.pallas import tpu as pltpu
from jax.sharding import NamedSharding, PartitionSpec as P

_B = 16384
_HIDDEN = 128
_NUM_CHARGES = 11
_CHUNK = 1024


def _fused_kernel(mz_ref, charge_ref, table_ref, w1_ref, b1_ref, w2_ref,
                  b2_ref, out_hbm, buf, sems):
    n_chunks = mz_ref.shape[0] // _CHUNK
    copies = []
    for m in range(n_chunks):
        rows = pl.ds(m * _CHUNK, _CHUNK)
        mz = mz_ref[rows]
        h = jnp.maximum(mz[:, None] * w1_ref[0][None, :] + b1_ref[:][None, :],
                        0.0)
        emb0 = jax.lax.dot_general(
            h, w2_ref[:],
            dimension_numbers=(((1,), (0,)), ((), ())),
            preferred_element_type=jnp.float32,
        ) + b2_ref[:][None, :]

        charge = charge_ref[rows]
        classes = jax.lax.broadcasted_iota(jnp.int32, (_CHUNK, _NUM_CHARGES),
                                           1)
        onehot = (charge[:, None] == classes).astype(jnp.float32)
        emb1 = jax.lax.dot_general(
            onehot, table_ref[:],
            dimension_numbers=(((1,), (0,)), ((), ())),
            preferred_element_type=jnp.float32,
        )

        buf[rows, :_HIDDEN] = emb0
        buf[rows, _HIDDEN:] = emb1

        copy = pltpu.make_async_copy(
            buf.at[rows, :], out_hbm.at[rows, :], sems.at[m])
        copy.start()
        copies.append(copy)
    for c in copies:
        c.wait()


def _shard_impl(precursor_mz, charge, charge_table, W1, b1, W2, b2):
    b_loc = precursor_mz.shape[0]
    return pl.pallas_call(
        _fused_kernel,
        out_specs=pl.BlockSpec(memory_space=pl.ANY),
        out_shape=jax.ShapeDtypeStruct((b_loc, 2 * _HIDDEN), jnp.float32),
        scratch_shapes=[
            pltpu.VMEM((b_loc, 2 * _HIDDEN), jnp.float32),
            pltpu.SemaphoreType.DMA((b_loc // _CHUNK,)),
        ],
    )(precursor_mz, charge, charge_table, W1, b1, W2, b2)


def kernel(precursor_mz, charge, charge_table, W1, b1, W2, b2):
    charge = charge.astype(jnp.int32)
    n_dev = jax.device_count()
    if _B % n_dev != 0 or (_B // n_dev) % _CHUNK != 0:
        n_dev = 1
    mesh = jax.make_mesh((n_dev,), ("d",))
    batch = NamedSharding(mesh, P("d"))
    repl = NamedSharding(mesh, P())
    args = (
        jax.reshard(precursor_mz, batch),
        jax.reshard(charge, batch),
        jax.reshard(charge_table, repl),
        jax.reshard(W1, repl),
        jax.reshard(b1, repl),
        jax.reshard(W2, repl),
        jax.reshard(b2, repl),
    )
    out = jax.shard_map(
        _shard_impl,
        mesh=mesh,
        in_specs=(P("d"), P("d"), P(), P(), P(), P(), P()),
        out_specs=P("d"),
        check_vma=False,
    )(*args)
    return out.reshape(_B, 2, _HIDDEN)


# speedup vs baseline: 10.3087x; 10.3087x over previous
import jax
import jax.numpy as jnp
from jax.experimental import pallas as pl
from jax.experimental.pallas import tpu as pltpu

_B = 16384
_HIDDEN = 128
_NUM_CHARGES = 11
_M = 16


def _fused_kernel(mz_ref, charge_ref, table_ref, w1_ref, b1_ref, w2_ref,
                  b2_ref, out_hbm, buf, sems):
    ch = _B // _M
    copies = []
    for m in range(_M):
        rows = pl.ds(m * ch, ch)
        mz = mz_ref[rows]
        h = jnp.maximum(mz[:, None] * w1_ref[0][None, :] + b1_ref[:][None, :],
                        0.0)
        emb0 = jax.lax.dot_general(
            h, w2_ref[:],
            dimension_numbers=(((1,), (0,)), ((), ())),
            preferred_element_type=jnp.float32,
        ) + b2_ref[:][None, :]

        charge = charge_ref[rows]
        classes = jax.lax.broadcasted_iota(jnp.int32, (ch, _NUM_CHARGES), 1)
        onehot = (charge[:, None] == classes).astype(jnp.float32)
        emb1 = jax.lax.dot_general(
            onehot, table_ref[:],
            dimension_numbers=(((1,), (0,)), ((), ())),
            preferred_element_type=jnp.float32,
        )

        buf[rows, :_HIDDEN] = emb0
        buf[rows, _HIDDEN:] = emb1

        copy = pltpu.make_async_copy(
            buf.at[rows, :], out_hbm.at[rows, :], sems.at[m])
        copy.start()
        copies.append(copy)
    for c in copies:
        c.wait()


@jax.jit
def kernel(precursor_mz, charge, charge_table, W1, b1, W2, b2):
    charge = charge.astype(jnp.int32)
    out = pl.pallas_call(
        _fused_kernel,
        out_specs=pl.BlockSpec(memory_space=pl.ANY),
        out_shape=jax.ShapeDtypeStruct((_B, 2 * _HIDDEN), jnp.float32),
        scratch_shapes=[
            pltpu.VMEM((_B, 2 * _HIDDEN), jnp.float32),
            pltpu.SemaphoreType.DMA((_M,)),
        ],
    )(precursor_mz, charge, charge_table, W1, b1, W2, b2)
    return out.reshape(_B, 2, _HIDDEN)
